# baseline (speedup 1.0000x reference)
import functools

import jax
import jax.numpy as jnp
from jax import lax
from jax.experimental import pallas as pl
from jax.experimental.pallas import tpu as pltpu
from jax.experimental.pallas import tpu_sc as plsc

_VOCAB = 1000000
_EMB_DIM = 64
_SEQ_LEN = 200
_BATCH = 4096
_PAD = 0

_B = _SEQ_LEN * _BATCH
_NW = 32
_PER_W = _B // _NW
_CHUNK = 128
_H = _CHUNK // 2
_NCHUNK = _PER_W // _CHUNK
_R = 4
_G = 2


def _emb_body(idx_hbm, w_pad, out128, idxall, bufE, bufO, *sems):
    gsemE = sems[:_R]
    gsemO = sems[_R:2 * _R]
    osem = sems[2 * _R:]
    wid = lax.axis_index("s") * 2 + lax.axis_index("c")
    chunk0 = wid * _NCHUNK
    obase = wid * (_PER_W // 2)

    pltpu.sync_copy(idx_hbm.at[pl.ds(chunk0, _NCHUNK)], idxall)

    def gather(k, slot):
        pltpu.async_copy(w_pad.at[idxall.at[k, pl.ds(0, _H)]],
                         bufE.at[slot], gsemE[slot])
        pltpu.async_copy(w_pad.at[idxall.at[k, pl.ds(_H, _H)]],
                         bufO.at[slot], gsemO[slot])

    def wait_gather(slot):
        pltpu.make_async_copy(w_pad.at[idxall.at[0, pl.ds(0, _H)]],
                              bufE.at[slot], gsemE[slot]).wait()
        pltpu.make_async_copy(w_pad.at[idxall.at[0, pl.ds(0, _H)]],
                              bufO.at[slot], gsemO[slot]).wait()

    def store(j, slot):
        dst = out128.at[pl.ds(obase + j * _H, _H)]
        pltpu.async_copy(bufE.at[slot, slice(None), pl.ds(0, _EMB_DIM)],
                         dst.at[slice(None), pl.ds(0, _EMB_DIM)], osem[slot])
        pltpu.async_copy(bufO.at[slot, slice(None), pl.ds(0, _EMB_DIM)],
                         dst.at[slice(None), pl.ds(_EMB_DIM, _EMB_DIM)],
                         osem[slot])

    def wait_store(slot):
        dst = out128.at[pl.ds(obase, _H)]
        for buf in (bufE, bufO):
            pltpu.make_async_copy(
                buf.at[slot, slice(None), pl.ds(0, _EMB_DIM)],
                dst.at[slice(None), pl.ds(0, _EMB_DIM)], osem[slot]).wait()

    for k in range(_G):
        gather(k, k)

    def group_body(g, carry):
        for s in range(_R):
            j = g * _R + s
            wait_gather(s)

            mn = idxall[j, pl.ds(0, 16)]
            for v in range(1, 8):
                mn = jnp.minimum(mn, idxall[j, pl.ds(v * 16, 16)])
            smin = mn[0]
            for l in range(1, 16):
                smin = jnp.minimum(smin, mn[l])

            @pl.when(smin == _PAD)
            def _fix(j=j, s=s):
                zeros16 = jnp.zeros((16,), jnp.float32)

                def zrow(buf, off):
                    def row_body(p, c2):
                        iv = idxall[j, pl.ds(off + p, 16)]

                        @pl.when(iv[0] == _PAD)
                        def _z():
                            for c in range(_EMB_DIM // 16):
                                buf[s, p, pl.ds(c * 16, 16)] = zeros16
                        return c2
                    lax.fori_loop(0, _H, row_body, 0)

                zrow(bufE, 0)
                zrow(bufO, _H)

            store(j, s)

            k = j + _G
            sk = (s + _G) % _R
            if s < _G:
                @pl.when(k >= _R)
                def _w(sk=sk):
                    wait_store(sk)
                gather(k, sk)
            else:
                @pl.when(k < _NCHUNK)
                def _wg(k=k, sk=sk):
                    wait_store(sk)
                    gather(k, sk)
        return carry

    lax.fori_loop(0, _NCHUNK // _R, group_body, 0)

    for s in range(_R):
        wait_store(s)


@functools.partial(jax.jit, static_argnums=())
def _emb(idx2d, w_pad):
    mesh = plsc.VectorSubcoreMesh(core_axis_name="c", subcore_axis_name="s")
    f = pl.kernel(
        _emb_body,
        out_type=jax.ShapeDtypeStruct((_B * _EMB_DIM // 128, 128),
                                      jnp.float32),
        mesh=mesh,
        scratch_types=[
            pltpu.VMEM((_NCHUNK, _CHUNK), jnp.int32),
            pltpu.VMEM((_R, _H, 128), jnp.float32),
            pltpu.VMEM((_R, _H, 128), jnp.float32),
        ] + [pltpu.SemaphoreType.DMA] * (3 * _R),
        compiler_params=pltpu.CompilerParams(use_tc_tiling_on_sc=False),
    )
    return f(idx2d, w_pad)


def kernel(src_input, W):
    idx3 = src_input.reshape(_B // _CHUNK, _H, 2)
    idx2d = jnp.concatenate([idx3[:, :, 0], idx3[:, :, 1]], axis=1)
    w_pad = jnp.pad(W, ((0, 0), (0, 128 - _EMB_DIM)))
    out = _emb(idx2d, w_pad)
    return out.reshape(_SEQ_LEN, _BATCH, _EMB_DIM)


# speedup vs baseline: 1.5542x; 1.5542x over previous
import functools

import jax
import jax.numpy as jnp
from jax import lax
from jax.experimental import pallas as pl
from jax.experimental.pallas import tpu as pltpu
from jax.experimental.pallas import tpu_sc as plsc

_VOCAB = 1000000
_EMB_DIM = 64
_SEQ_LEN = 200
_BATCH = 4096
_PAD = 0

_B = _SEQ_LEN * _BATCH
_NW = 32
_PER_W = _B // _NW
_CHUNK = 128
_NCHUNK = _PER_W // _CHUNK
_R = 8
_G = 4


def _emb_body(idx_hbm, w_hbm, out_hbm, idxall, rows, *sems):
    gsem = sems[:_R]
    osem = sems[_R:]
    wid = lax.axis_index("s") * 2 + lax.axis_index("c")
    chunk0 = wid * _NCHUNK
    obase = wid * _PER_W

    pltpu.sync_copy(idx_hbm.at[pl.ds(chunk0, _NCHUNK)], idxall)

    def gather(k, slot):
        return pltpu.async_copy(w_hbm.at[idxall.at[k]], rows.at[slot],
                                gsem[slot])

    def wait_gather(slot):
        pltpu.make_async_copy(w_hbm.at[idxall.at[0]], rows.at[slot],
                              gsem[slot]).wait()

    def wait_store(slot):
        pltpu.make_async_copy(rows.at[slot], out_hbm.at[pl.ds(obase, _CHUNK)],
                              osem[slot]).wait()

    for k in range(_G):
        gather(k, k)

    def group_body(g, carry):
        for s in range(_R):
            j = g * _R + s
            wait_gather(s)

            mn = idxall[j, pl.ds(0, 16)]
            for v in range(1, 8):
                mn = jnp.minimum(mn, idxall[j, pl.ds(v * 16, 16)])
            smin = mn[0]
            for l in range(1, 16):
                smin = jnp.minimum(smin, mn[l])

            @pl.when(smin == _PAD)
            def _fix(j=j, s=s):
                zeros16 = jnp.zeros((16,), jnp.float32)

                def row_body(r, c2):
                    iv = idxall[j, pl.ds(r, 16)]

                    @pl.when(iv[0] == _PAD)
                    def _z():
                        for c in range(_EMB_DIM // 16):
                            rows[s, r, pl.ds(c * 16, 16)] = zeros16
                    return c2

                lax.fori_loop(0, _CHUNK, row_body, 0)

            pltpu.async_copy(rows.at[s], out_hbm.at[pl.ds(obase + j * _CHUNK,
                                                          _CHUNK)], osem[s])

            k = j + _G
            sk = (s + _G) % _R
            if s < _G:
                @pl.when(k >= _R)
                def _w(sk=sk):
                    wait_store(sk)
                gather(k, sk)
            else:
                @pl.when(k < _NCHUNK)
                def _wg(k=k, sk=sk):
                    wait_store(sk)
                    gather(k, sk)
        return carry

    lax.fori_loop(0, _NCHUNK // _R, group_body, 0)

    for s in range(_R):
        wait_store(s)


@functools.partial(jax.jit, static_argnums=())
def _emb(idx2d, w):
    mesh = plsc.VectorSubcoreMesh(core_axis_name="c", subcore_axis_name="s")
    f = pl.kernel(
        _emb_body,
        out_type=jax.ShapeDtypeStruct((_B, _EMB_DIM), jnp.float32),
        mesh=mesh,
        scratch_types=[
            pltpu.VMEM((_NCHUNK, _CHUNK), jnp.int32),
            pltpu.VMEM((_R, _CHUNK, _EMB_DIM), jnp.float32),
        ] + [pltpu.SemaphoreType.DMA] * (2 * _R),
        compiler_params=pltpu.CompilerParams(use_tc_tiling_on_sc=False),
    )
    return f(idx2d, w)


def kernel(src_input, W):
    idx2d = src_input.reshape(_B // _CHUNK, _CHUNK)
    out = _emb(idx2d, W)
    return out.reshape(_SEQ_LEN, _BATCH, _EMB_DIM)
